# Initial kernel scaffold; baseline (speedup 1.0000x reference)
import jax
import jax.numpy as jnp
from jax.experimental import pallas as pl


def kernel(features, adj_lst, W_heads, a1_heads, a2_heads, W_out, a1_out, a2_out):
    raise NotImplementedError("write your pallas kernel here")


# speedup vs baseline: 16.1189x; 16.1189x over previous
import functools

import jax
import jax.numpy as jnp
from jax.experimental import pallas as pl

_BLK = 256
_NEG = -1e9
_SLOPE = 0.2


def _lrelu(x):
    return jnp.where(x >= 0, x, _SLOPE * x)


def _proj_kernel(x_ref, w_ref, a1_ref, a2_ref, h_ref, f1_ref, f2_ref):
    h = jnp.dot(x_ref[...], w_ref[...], preferred_element_type=jnp.float32)
    h_ref[...] = h
    f1_ref[...] = jnp.dot(h, a1_ref[...], preferred_element_type=jnp.float32)
    f2_ref[...] = jnp.dot(h, a2_ref[...], preferred_element_type=jnp.float32)


def _proj(x, w, a1, a2):
    n, d_in = x.shape
    d_out = w.shape[1]
    nh = a1.shape[1]
    return pl.pallas_call(
        _proj_kernel,
        grid=(n // _BLK,),
        in_specs=[
            pl.BlockSpec((_BLK, d_in), lambda i: (i, 0)),
            pl.BlockSpec((d_in, d_out), lambda i: (0, 0)),
            pl.BlockSpec((d_out, nh), lambda i: (0, 0)),
            pl.BlockSpec((d_out, nh), lambda i: (0, 0)),
        ],
        out_specs=[
            pl.BlockSpec((_BLK, d_out), lambda i: (i, 0)),
            pl.BlockSpec((_BLK, nh), lambda i: (i, 0)),
            pl.BlockSpec((_BLK, nh), lambda i: (i, 0)),
        ],
        out_shape=[
            jax.ShapeDtypeStruct((n, d_out), jnp.float32),
            jax.ShapeDtypeStruct((n, nh), jnp.float32),
            jax.ShapeDtypeStruct((n, nh), jnp.float32),
        ],
    )(x, w, a1, a2)


def _rank_kernel(f2_ref, f2t_ref, rank_ref, *, nh, n):
    i = pl.program_id(0)
    idx_all = jax.lax.broadcasted_iota(jnp.int32, (1, n), 1)
    idx_me = i * _BLK + jax.lax.broadcasted_iota(jnp.int32, (_BLK, 1), 0)
    for hd in range(nh):
        me = f2_ref[:, hd : hd + 1]
        alln = f2t_ref[hd : hd + 1, :]
        gt = (alln > me).astype(jnp.int32)
        eq = ((alln == me) & (idx_all < idx_me)).astype(jnp.int32)
        rank_ref[:, hd : hd + 1] = jnp.sum(gt + eq, axis=1, keepdims=True)


def _ranks(f2, f2t):
    n, nh = f2.shape
    return pl.pallas_call(
        functools.partial(_rank_kernel, nh=nh, n=n),
        grid=(n // _BLK,),
        in_specs=[
            pl.BlockSpec((_BLK, nh), lambda i: (i, 0)),
            pl.BlockSpec((nh, n), lambda i: (0, 0)),
        ],
        out_specs=pl.BlockSpec((_BLK, nh), lambda i: (i, 0)),
        out_shape=jax.ShapeDtypeStruct((n, nh), jnp.int32),
    )(f2, f2t)


def _attn_kernel(adj_ref, f1_ref, f2t_ref, rankt_ref, h_ref, o_ref, *, nh, hid, k, n, elu):
    adjb = adj_ref[...] > 0
    m = jnp.sum(adjb.astype(jnp.int32), axis=1, keepdims=True)
    feas = m >= k
    nbits = max(1, (n - 1).bit_length())
    for hd in range(nh):
        f2row = f2t_ref[hd : hd + 1, :]
        rankrow = rankt_ref[hd : hd + 1, :]
        f1col = f1_ref[:, hd : hd + 1]
        lo = jnp.zeros((adjb.shape[0], 1), jnp.int32)
        hi = jnp.full((adjb.shape[0], 1), n - 1, jnp.int32)
        for _ in range(nbits):
            mid = (lo + hi) >> 1
            cnt = jnp.sum(
                (adjb & (rankrow <= mid)).astype(jnp.int32), axis=1, keepdims=True
            )
            ok = cnt >= k
            lo = jnp.where(ok, lo, mid + 1)
            hi = jnp.where(ok, mid, hi)
        vstar = jnp.sum(
            jnp.where(rankrow == lo, f2row, 0.0), axis=1, keepdims=True
        )
        e = jnp.where(adjb, _lrelu(f1col + f2row), _NEG)
        thr = jnp.where(feas, _lrelu(f1col + vstar), _NEG)
        e = jnp.where(e >= thr, e, _NEG)
        mx = jnp.max(e, axis=1, keepdims=True)
        p = jnp.exp(e - mx)
        att = p / jnp.sum(p, axis=1, keepdims=True)
        out = jnp.dot(
            att, h_ref[:, hd * hid : (hd + 1) * hid], preferred_element_type=jnp.float32
        )
        if elu:
            out = jnp.where(out > 0, out, jnp.exp(jnp.minimum(out, 0.0)) - 1.0)
        o_ref[:, hd * hid : (hd + 1) * hid] = out


def _attention(adj, f1, f2t, rankt, h, *, k, elu):
    n = adj.shape[0]
    nh = f1.shape[1]
    hid = h.shape[1] // nh
    return pl.pallas_call(
        functools.partial(_attn_kernel, nh=nh, hid=hid, k=k, n=n, elu=elu),
        grid=(n // _BLK,),
        in_specs=[
            pl.BlockSpec((_BLK, n), lambda i: (i, 0)),
            pl.BlockSpec((_BLK, nh), lambda i: (i, 0)),
            pl.BlockSpec((nh, n), lambda i: (0, 0)),
            pl.BlockSpec((nh, n), lambda i: (0, 0)),
            pl.BlockSpec((n, h.shape[1]), lambda i: (0, 0)),
        ],
        out_specs=pl.BlockSpec((_BLK, h.shape[1]), lambda i: (i, 0)),
        out_shape=jax.ShapeDtypeStruct((n, h.shape[1]), jnp.float32),
    )(adj, f1, f2t, rankt, h)


def kernel(features, adj_lst, W_heads, a1_heads, a2_heads, W_out, a1_out, a2_out):
    n = features.shape[0]
    n_layers, n_heads = W_heads.shape[0], W_heads.shape[1]
    hid = W_heads.shape[3]
    k = int(0.5 * n)
    eye = jnp.eye(n_heads, dtype=jnp.float32)

    x = features
    for i in range(n_layers):
        adj = adj_lst[i]
        w_cat = jnp.transpose(W_heads[i], (1, 0, 2)).reshape(
            W_heads.shape[2], n_heads * hid
        )
        a1c = (eye[:, None, :] * a1_heads[i, :, :, 0][:, :, None]).reshape(
            n_heads * hid, n_heads
        )
        a2c = (eye[:, None, :] * a2_heads[i, :, :, 0][:, :, None]).reshape(
            n_heads * hid, n_heads
        )
        h, f1, f2 = _proj(x, w_cat, a1c, a2c)
        f2t = f2.T
        rankt = _ranks(f2, f2t).T
        hcat = _attention(adj, f1, f2t, rankt, h, k=k, elu=False)

        h2, f1o, f2o = _proj(hcat, W_out[i], a1_out[i], a2_out[i])
        f2ot = f2o.T
        rankot = _ranks(f2o, f2ot).T
        x = _attention(adj, f1o, f2ot, rankot, h2, k=k, elu=True)
    return x


# speedup vs baseline: 24.9436x; 1.5475x over previous
import functools

import jax
import jax.numpy as jnp
from jax.experimental import pallas as pl

_BLK = 256
_NEG = -1e9
_SLOPE = 0.2


def _lrelu(x):
    return jnp.where(x >= 0, x, _SLOPE * x)


def _proj_kernel(x_ref, w_ref, a1_ref, a2_ref, h_ref, f1_ref, f2_ref):
    h = jnp.dot(x_ref[...], w_ref[...], preferred_element_type=jnp.float32)
    h_ref[...] = h
    f1_ref[...] = jnp.dot(h, a1_ref[...], preferred_element_type=jnp.float32)
    f2_ref[...] = jnp.dot(h, a2_ref[...], preferred_element_type=jnp.float32)


def _proj(x, w, a1, a2):
    n, d_in = x.shape
    d_out = w.shape[1]
    nh = a1.shape[1]
    return pl.pallas_call(
        _proj_kernel,
        grid=(n // _BLK,),
        in_specs=[
            pl.BlockSpec((_BLK, d_in), lambda i: (i, 0)),
            pl.BlockSpec((d_in, d_out), lambda i: (0, 0)),
            pl.BlockSpec((d_out, nh), lambda i: (0, 0)),
            pl.BlockSpec((d_out, nh), lambda i: (0, 0)),
        ],
        out_specs=[
            pl.BlockSpec((_BLK, d_out), lambda i: (i, 0)),
            pl.BlockSpec((_BLK, nh), lambda i: (i, 0)),
            pl.BlockSpec((_BLK, nh), lambda i: (i, 0)),
        ],
        out_shape=[
            jax.ShapeDtypeStruct((n, d_out), jnp.float32),
            jax.ShapeDtypeStruct((n, nh), jnp.float32),
            jax.ShapeDtypeStruct((n, nh), jnp.float32),
        ],
    )(x, w, a1, a2)


_NBUCKET = 128


def _rank_kernel(f2_ref, f2t_ref, rank_ref, scum_ref, *, nh, n):
    i = pl.program_id(0)
    idx_all = jax.lax.broadcasted_iota(jnp.int32, (1, n), 1)
    idx_me = i * _BLK + jax.lax.broadcasted_iota(jnp.int32, (_BLK, 1), 0)
    bw = n // _NBUCKET
    bucket_hi = bw * jax.lax.broadcasted_iota(jnp.int32, (1, _NBUCKET), 1) + (bw - 1)
    for hd in range(nh):
        me = f2_ref[:, hd : hd + 1]
        alln = f2t_ref[hd : hd + 1, :]
        gt = (alln > me).astype(jnp.int32)
        eq = ((alln == me) & (idx_all < idx_me)).astype(jnp.int32)
        rank = jnp.sum(gt + eq, axis=1, keepdims=True)
        rank_ref[:, hd : hd + 1] = rank
        scum_ref[:, hd * _NBUCKET : (hd + 1) * _NBUCKET] = (
            rank <= bucket_hi
        ).astype(jnp.float32)


def _ranks(f2, f2t):
    n, nh = f2.shape
    return pl.pallas_call(
        functools.partial(_rank_kernel, nh=nh, n=n),
        grid=(n // _BLK,),
        in_specs=[
            pl.BlockSpec((_BLK, nh), lambda i: (i, 0)),
            pl.BlockSpec((nh, n), lambda i: (0, 0)),
        ],
        out_specs=[
            pl.BlockSpec((_BLK, nh), lambda i: (i, 0)),
            pl.BlockSpec((_BLK, nh * _NBUCKET), lambda i: (i, 0)),
        ],
        out_shape=[
            jax.ShapeDtypeStruct((n, nh), jnp.int32),
            jax.ShapeDtypeStruct((n, nh * _NBUCKET), jnp.float32),
        ],
    )(f2, f2t)


def _attn_kernel(
    adj_ref, f1_ref, f2t_ref, rankt_ref, scum_ref, h_ref, o_ref, *, nh, hid, k, n, elu
):
    adjb = adj_ref[...] > 0
    adjf = adjb.astype(jnp.float32)
    m = jnp.sum(adjf, axis=1, keepdims=True)
    feas = m >= float(k)
    bw = n // _NBUCKET
    nbits = max(1, (bw - 1).bit_length())
    for hd in range(nh):
        f2row = f2t_ref[hd : hd + 1, :]
        rankrow = rankt_ref[hd : hd + 1, :]
        f1col = f1_ref[:, hd : hd + 1]
        ccnt = jnp.dot(
            adjf,
            scum_ref[:, hd * _NBUCKET : (hd + 1) * _NBUCKET],
            preferred_element_type=jnp.float32,
        )
        bstar = jnp.sum(
            (ccnt < float(k)).astype(jnp.int32), axis=1, keepdims=True
        )
        lo = jnp.minimum(bstar, _NBUCKET - 1) * bw
        hi = lo + bw - 1
        for _ in range(nbits):
            mid = (lo + hi) >> 1
            cnt = jnp.sum(
                (adjb & (rankrow <= mid)).astype(jnp.int32), axis=1, keepdims=True
            )
            ok = cnt >= k
            lo = jnp.where(ok, lo, mid + 1)
            hi = jnp.where(ok, mid, hi)
        vstar = jnp.sum(
            jnp.where(rankrow == lo, f2row, 0.0), axis=1, keepdims=True
        )
        e = jnp.where(adjb, _lrelu(f1col + f2row), _NEG)
        thr = jnp.where(feas, _lrelu(f1col + vstar), _NEG)
        e = jnp.where(e >= thr, e, _NEG)
        mx = jnp.max(e, axis=1, keepdims=True)
        p = jnp.exp(e - mx)
        att = p / jnp.sum(p, axis=1, keepdims=True)
        out = jnp.dot(
            att, h_ref[:, hd * hid : (hd + 1) * hid], preferred_element_type=jnp.float32
        )
        if elu:
            out = jnp.where(out > 0, out, jnp.exp(jnp.minimum(out, 0.0)) - 1.0)
        o_ref[:, hd * hid : (hd + 1) * hid] = out


def _attention(adj, f1, f2t, rankt, scum, h, *, k, elu):
    n = adj.shape[0]
    nh = f1.shape[1]
    hid = h.shape[1] // nh
    return pl.pallas_call(
        functools.partial(_attn_kernel, nh=nh, hid=hid, k=k, n=n, elu=elu),
        grid=(n // _BLK,),
        in_specs=[
            pl.BlockSpec((_BLK, n), lambda i: (i, 0)),
            pl.BlockSpec((_BLK, nh), lambda i: (i, 0)),
            pl.BlockSpec((nh, n), lambda i: (0, 0)),
            pl.BlockSpec((nh, n), lambda i: (0, 0)),
            pl.BlockSpec((n, nh * _NBUCKET), lambda i: (0, 0)),
            pl.BlockSpec((n, h.shape[1]), lambda i: (0, 0)),
        ],
        out_specs=pl.BlockSpec((_BLK, h.shape[1]), lambda i: (i, 0)),
        out_shape=jax.ShapeDtypeStruct((n, h.shape[1]), jnp.float32),
    )(adj, f1, f2t, rankt, scum, h)


def kernel(features, adj_lst, W_heads, a1_heads, a2_heads, W_out, a1_out, a2_out):
    n = features.shape[0]
    n_layers, n_heads = W_heads.shape[0], W_heads.shape[1]
    hid = W_heads.shape[3]
    k = int(0.5 * n)
    eye = jnp.eye(n_heads, dtype=jnp.float32)

    x = features
    for i in range(n_layers):
        adj = adj_lst[i]
        w_cat = jnp.transpose(W_heads[i], (1, 0, 2)).reshape(
            W_heads.shape[2], n_heads * hid
        )
        a1c = (eye[:, None, :] * a1_heads[i, :, :, 0][:, :, None]).reshape(
            n_heads * hid, n_heads
        )
        a2c = (eye[:, None, :] * a2_heads[i, :, :, 0][:, :, None]).reshape(
            n_heads * hid, n_heads
        )
        h, f1, f2 = _proj(x, w_cat, a1c, a2c)
        f2t = f2.T
        rank, scum = _ranks(f2, f2t)
        hcat = _attention(adj, f1, f2t, rank.T, scum, h, k=k, elu=False)

        h2, f1o, f2o = _proj(hcat, W_out[i], a1_out[i], a2_out[i])
        f2ot = f2o.T
        ranko, scumo = _ranks(f2o, f2ot)
        x = _attention(adj, f1o, f2ot, ranko.T, scumo, h2, k=k, elu=True)
    return x


# speedup vs baseline: 28.3395x; 1.1361x over previous
import functools

import jax
import jax.numpy as jnp
from jax.experimental import pallas as pl

_BLK = 256
_NBUCKET = 128
_NEG = -1e9
_SLOPE = 0.2


def _lrelu(x):
    return jnp.where(x >= 0, x, _SLOPE * x)


def _proj_kernel(x_ref, w_ref, a1_ref, a2_ref, h_ref, f1_ref, f2_ref):
    h = jnp.dot(x_ref[...], w_ref[...], preferred_element_type=jnp.float32)
    h_ref[...] = h
    f1_ref[...] = jnp.dot(h, a1_ref[...], preferred_element_type=jnp.float32)
    f2_ref[...] = jnp.dot(h, a2_ref[...], preferred_element_type=jnp.float32)


def _proj(x, w, a1, a2):
    n, d_in = x.shape
    d_out = w.shape[1]
    nh = a1.shape[1]
    return pl.pallas_call(
        _proj_kernel,
        grid=(n // _BLK,),
        in_specs=[
            pl.BlockSpec((_BLK, d_in), lambda i: (i, 0)),
            pl.BlockSpec((d_in, d_out), lambda i: (0, 0)),
            pl.BlockSpec((d_out, nh), lambda i: (0, 0)),
            pl.BlockSpec((d_out, nh), lambda i: (0, 0)),
        ],
        out_specs=[
            pl.BlockSpec((_BLK, d_out), lambda i: (i, 0)),
            pl.BlockSpec((_BLK, nh), lambda i: (i, 0)),
            pl.BlockSpec((_BLK, nh), lambda i: (i, 0)),
        ],
        out_shape=[
            jax.ShapeDtypeStruct((n, d_out), jnp.float32),
            jax.ShapeDtypeStruct((n, nh), jnp.float32),
            jax.ShapeDtypeStruct((n, nh), jnp.float32),
        ],
    )(x, w, a1, a2)


def _rank_kernel(f2_ref, f2t_ref, rank_ref, scum_ref, *, nh, n):
    bw = n // _NBUCKET
    bucket_hi = bw * jax.lax.broadcasted_iota(jnp.int32, (1, _NBUCKET), 1) + (bw - 1)
    for hd in range(nh):
        me = f2_ref[:, hd : hd + 1]
        alln = f2t_ref[hd : hd + 1, :]
        rank = jnp.sum((alln > me).astype(jnp.int32), axis=1, keepdims=True)
        rank_ref[:, hd : hd + 1] = rank
        scum_ref[:, hd * _NBUCKET : (hd + 1) * _NBUCKET] = (rank <= bucket_hi).astype(
            jnp.bfloat16
        )


def _ranks(f2, f2t):
    n, nh = f2.shape
    return pl.pallas_call(
        functools.partial(_rank_kernel, nh=nh, n=n),
        grid=(n // _BLK,),
        in_specs=[
            pl.BlockSpec((_BLK, nh), lambda i: (i, 0)),
            pl.BlockSpec((nh, n), lambda i: (0, 0)),
        ],
        out_specs=[
            pl.BlockSpec((_BLK, nh), lambda i: (i, 0)),
            pl.BlockSpec((_BLK, nh * _NBUCKET), lambda i: (i, 0)),
        ],
        out_shape=[
            jax.ShapeDtypeStruct((n, nh), jnp.int32),
            jax.ShapeDtypeStruct((n, nh * _NBUCKET), jnp.bfloat16),
        ],
    )(f2, f2t)


def _attn_kernel(
    adj_ref, f1_ref, f2t_ref, rankt_ref, scum_ref, h_ref, o_ref, *, nh, hid, k, n, elu
):
    adjb = adj_ref[...] > 0
    adjh = adjb.astype(jnp.bfloat16)
    bw = n // _NBUCKET
    nbits = max(1, (bw - 1).bit_length())
    for hd in range(nh):
        f2row = f2t_ref[hd : hd + 1, :]
        rankrow = rankt_ref[hd : hd + 1, :]
        f1col = f1_ref[:, hd : hd + 1]
        ccnt = jnp.dot(
            adjh,
            scum_ref[:, hd * _NBUCKET : (hd + 1) * _NBUCKET],
            preferred_element_type=jnp.float32,
        )
        feas = ccnt[:, _NBUCKET - 1 :] >= float(k)
        bstar = jnp.sum((ccnt < float(k)).astype(jnp.int32), axis=1, keepdims=True)
        lo = jnp.minimum(bstar, _NBUCKET - 1) * bw
        hi = lo + bw - 1
        for _ in range(nbits):
            mid = (lo + hi) >> 1
            cnt = jnp.sum(
                (adjb & (rankrow <= mid)).astype(jnp.int32), axis=1, keepdims=True
            )
            ok = cnt >= k
            lo = jnp.where(ok, lo, mid + 1)
            hi = jnp.where(ok, mid, hi)
        vstar = jnp.max(
            jnp.where(rankrow == lo, jnp.broadcast_to(f2row, rankrow.shape), -3e38),
            axis=1,
            keepdims=True,
        )
        raw = f1col + f2row
        lr = _lrelu(raw)
        thr = jnp.where(feas, _lrelu(f1col + vstar), _NEG)
        e = jnp.where(adjb & (lr >= thr), lr, _NEG)
        mx = jnp.max(e, axis=1, keepdims=True)
        p = jnp.exp(e - mx)
        s = jnp.sum(p, axis=1, keepdims=True)
        out = jnp.dot(
            p, h_ref[:, hd * hid : (hd + 1) * hid], preferred_element_type=jnp.float32
        )
        out = out / s
        if elu:
            out = jnp.where(out > 0, out, jnp.exp(jnp.minimum(out, 0.0)) - 1.0)
        o_ref[:, hd * hid : (hd + 1) * hid] = out


def _attention(adj, f1, f2t, rankt, scum, h, *, k, elu):
    n = adj.shape[0]
    nh = f1.shape[1]
    hid = h.shape[1] // nh
    return pl.pallas_call(
        functools.partial(_attn_kernel, nh=nh, hid=hid, k=k, n=n, elu=elu),
        grid=(n // _BLK,),
        in_specs=[
            pl.BlockSpec((_BLK, n), lambda i: (i, 0)),
            pl.BlockSpec((_BLK, nh), lambda i: (i, 0)),
            pl.BlockSpec((nh, n), lambda i: (0, 0)),
            pl.BlockSpec((nh, n), lambda i: (0, 0)),
            pl.BlockSpec((n, nh * _NBUCKET), lambda i: (0, 0)),
            pl.BlockSpec((n, h.shape[1]), lambda i: (0, 0)),
        ],
        out_specs=pl.BlockSpec((_BLK, h.shape[1]), lambda i: (i, 0)),
        out_shape=jax.ShapeDtypeStruct((n, h.shape[1]), jnp.float32),
    )(adj, f1, f2t, rankt, scum, h)


def kernel(features, adj_lst, W_heads, a1_heads, a2_heads, W_out, a1_out, a2_out):
    n = features.shape[0]
    n_layers, n_heads = W_heads.shape[0], W_heads.shape[1]
    hid = W_heads.shape[3]
    k = int(0.5 * n)
    eye = jnp.eye(n_heads, dtype=jnp.float32)

    x = features
    for i in range(n_layers):
        adj = adj_lst[i]
        w_cat = jnp.transpose(W_heads[i], (1, 0, 2)).reshape(
            W_heads.shape[2], n_heads * hid
        )
        a1c = (eye[:, None, :] * a1_heads[i, :, :, 0][:, :, None]).reshape(
            n_heads * hid, n_heads
        )
        a2c = (eye[:, None, :] * a2_heads[i, :, :, 0][:, :, None]).reshape(
            n_heads * hid, n_heads
        )
        h, f1, f2 = _proj(x, w_cat, a1c, a2c)
        f2t = f2.T
        rank, scum = _ranks(f2, f2t)
        hcat = _attention(adj, f1, f2t, rank.T, scum, h, k=k, elu=False)

        h2, f1o, f2o = _proj(hcat, W_out[i], a1_out[i], a2_out[i])
        f2ot = f2o.T
        ranko, scumo = _ranks(f2o, f2ot)
        x = _attention(adj, f1o, f2ot, ranko.T, scumo, h2, k=k, elu=True)
    return x


# speedup vs baseline: 37.7073x; 1.3306x over previous
import functools

import jax
import jax.numpy as jnp
from jax.experimental import pallas as pl

_BLK = 256
_NBUCKET = 128
_NEG = -1e9
_SLOPE = 0.2


def _lrelu(x):
    return jnp.where(x >= 0, x, _SLOPE * x)


def _proj_kernel(x_ref, w_ref, a1_ref, a2_ref, h_ref, f1_ref, f2_ref):
    h = jnp.dot(x_ref[...], w_ref[...], preferred_element_type=jnp.float32)
    h_ref[...] = h
    f1_ref[...] = jnp.dot(h, a1_ref[...], preferred_element_type=jnp.float32)
    f2_ref[...] = jnp.dot(h, a2_ref[...], preferred_element_type=jnp.float32)


def _proj(x, w, a1, a2):
    n, d_in = x.shape
    d_out = w.shape[1]
    nh = a1.shape[1]
    return pl.pallas_call(
        _proj_kernel,
        grid=(n // _BLK,),
        in_specs=[
            pl.BlockSpec((_BLK, d_in), lambda i: (i, 0)),
            pl.BlockSpec((d_in, d_out), lambda i: (0, 0)),
            pl.BlockSpec((d_out, nh), lambda i: (0, 0)),
            pl.BlockSpec((d_out, nh), lambda i: (0, 0)),
        ],
        out_specs=[
            pl.BlockSpec((_BLK, d_out), lambda i: (i, 0)),
            pl.BlockSpec((_BLK, nh), lambda i: (i, 0)),
            pl.BlockSpec((_BLK, nh), lambda i: (i, 0)),
        ],
        out_shape=[
            jax.ShapeDtypeStruct((n, d_out), jnp.float32),
            jax.ShapeDtypeStruct((n, nh), jnp.float32),
            jax.ShapeDtypeStruct((n, nh), jnp.float32),
        ],
    )(x, w, a1, a2)


def _rank_kernel(f2_ref, f2t_ref, rank_ref, scum_ref, smod_ref, *, nh, n):
    bw = n // _NBUCKET
    bucket_hi = bw * jax.lax.broadcasted_iota(jnp.int32, (1, _NBUCKET), 1) + (bw - 1)
    mod_t = jax.lax.broadcasted_iota(jnp.int32, (1, bw), 1)
    for hd in range(nh):
        me = f2_ref[:, hd : hd + 1]
        alln = f2t_ref[hd : hd + 1, :]
        rank = jnp.sum((alln > me).astype(jnp.int32), axis=1, keepdims=True)
        rank_ref[:, hd : hd + 1] = rank
        scum_ref[:, hd * _NBUCKET : (hd + 1) * _NBUCKET] = (rank <= bucket_hi).astype(
            jnp.bfloat16
        )
        smod_ref[:, hd * bw : (hd + 1) * bw] = ((rank & (bw - 1)) <= mod_t).astype(
            jnp.bfloat16
        )


def _ranks(f2, f2t):
    n, nh = f2.shape
    bw = n // _NBUCKET
    return pl.pallas_call(
        functools.partial(_rank_kernel, nh=nh, n=n),
        grid=(n // _BLK,),
        in_specs=[
            pl.BlockSpec((_BLK, nh), lambda i: (i, 0)),
            pl.BlockSpec((nh, n), lambda i: (0, 0)),
        ],
        out_specs=[
            pl.BlockSpec((_BLK, nh), lambda i: (i, 0)),
            pl.BlockSpec((_BLK, nh * _NBUCKET), lambda i: (i, 0)),
            pl.BlockSpec((_BLK, nh * bw), lambda i: (i, 0)),
        ],
        out_shape=[
            jax.ShapeDtypeStruct((n, nh), jnp.int32),
            jax.ShapeDtypeStruct((n, nh * _NBUCKET), jnp.bfloat16),
            jax.ShapeDtypeStruct((n, nh * bw), jnp.bfloat16),
        ],
    )(f2, f2t)


def _attn_kernel(
    adj_ref,
    f1_ref,
    f2t_ref,
    rankt_ref,
    scum_ref,
    smod_ref,
    h_ref,
    o_ref,
    *,
    nh,
    hid,
    k,
    n,
    elu,
):
    adjb = adj_ref[...] > 0
    adjh = adjb.astype(jnp.bfloat16)
    bw = n // _NBUCKET
    shift = (bw - 1).bit_length()
    for hd in range(nh):
        f2row = f2t_ref[hd : hd + 1, :]
        rankrow = rankt_ref[hd : hd + 1, :]
        divrow = rankrow >> shift
        f1col = f1_ref[:, hd : hd + 1]
        ccnt = jnp.dot(
            adjh,
            scum_ref[:, hd * _NBUCKET : (hd + 1) * _NBUCKET],
            preferred_element_type=jnp.float32,
        )
        feas = ccnt[:, _NBUCKET - 1 :] >= float(k)
        bstar = jnp.sum((ccnt < float(k)).astype(jnp.int32), axis=1, keepdims=True)
        bstar = jnp.minimum(bstar, _NBUCKET - 1)
        cprev = jnp.max(
            jnp.where(ccnt < float(k), ccnt, 0.0), axis=1, keepdims=True
        )
        inb = (adjb & (divrow == bstar)).astype(jnp.bfloat16)
        cumf = jnp.dot(
            inb,
            smod_ref[:, hd * bw : (hd + 1) * bw],
            preferred_element_type=jnp.float32,
        )
        tstar = jnp.sum(
            (cprev + cumf < float(k)).astype(jnp.int32), axis=1, keepdims=True
        )
        lo = bstar * bw + jnp.minimum(tstar, bw - 1)
        vstar = jnp.max(
            jnp.where(rankrow == lo, jnp.broadcast_to(f2row, rankrow.shape), -3e38),
            axis=1,
            keepdims=True,
        )
        raw = f1col + f2row
        lr = _lrelu(raw)
        thr = jnp.where(feas, _lrelu(f1col + vstar), _NEG)
        e = jnp.where(adjb & (lr >= thr), lr, _NEG)
        mx = jnp.max(e, axis=1, keepdims=True)
        p = jnp.exp(e - mx)
        s = jnp.sum(p, axis=1, keepdims=True)
        out = jnp.dot(
            p, h_ref[:, hd * hid : (hd + 1) * hid], preferred_element_type=jnp.float32
        )
        out = out / s
        if elu:
            out = jnp.where(out > 0, out, jnp.exp(jnp.minimum(out, 0.0)) - 1.0)
        o_ref[:, hd * hid : (hd + 1) * hid] = out


def _attention(adj, f1, f2t, rankt, scum, smod, h, *, k, elu):
    n = adj.shape[0]
    nh = f1.shape[1]
    hid = h.shape[1] // nh
    bw = n // _NBUCKET
    return pl.pallas_call(
        functools.partial(_attn_kernel, nh=nh, hid=hid, k=k, n=n, elu=elu),
        grid=(n // _BLK,),
        in_specs=[
            pl.BlockSpec((_BLK, n), lambda i: (i, 0)),
            pl.BlockSpec((_BLK, nh), lambda i: (i, 0)),
            pl.BlockSpec((nh, n), lambda i: (0, 0)),
            pl.BlockSpec((nh, n), lambda i: (0, 0)),
            pl.BlockSpec((n, nh * _NBUCKET), lambda i: (0, 0)),
            pl.BlockSpec((n, nh * bw), lambda i: (0, 0)),
            pl.BlockSpec((n, h.shape[1]), lambda i: (0, 0)),
        ],
        out_specs=pl.BlockSpec((_BLK, h.shape[1]), lambda i: (i, 0)),
        out_shape=jax.ShapeDtypeStruct((n, h.shape[1]), jnp.float32),
    )(adj, f1, f2t, rankt, scum, smod, h)


def kernel(features, adj_lst, W_heads, a1_heads, a2_heads, W_out, a1_out, a2_out):
    n = features.shape[0]
    n_layers, n_heads = W_heads.shape[0], W_heads.shape[1]
    hid = W_heads.shape[3]
    k = int(0.5 * n)
    eye = jnp.eye(n_heads, dtype=jnp.float32)

    x = features
    for i in range(n_layers):
        adj = adj_lst[i]
        w_cat = jnp.transpose(W_heads[i], (1, 0, 2)).reshape(
            W_heads.shape[2], n_heads * hid
        )
        a1c = (eye[:, None, :] * a1_heads[i, :, :, 0][:, :, None]).reshape(
            n_heads * hid, n_heads
        )
        a2c = (eye[:, None, :] * a2_heads[i, :, :, 0][:, :, None]).reshape(
            n_heads * hid, n_heads
        )
        h, f1, f2 = _proj(x, w_cat, a1c, a2c)
        f2t = f2.T
        rank, scum, smod = _ranks(f2, f2t)
        hcat = _attention(adj, f1, f2t, rank.T, scum, smod, h, k=k, elu=False)

        h2, f1o, f2o = _proj(hcat, W_out[i], a1_out[i], a2_out[i])
        f2ot = f2o.T
        ranko, scumo, smodo = _ranks(f2o, f2ot)
        x = _attention(adj, f1o, f2ot, ranko.T, scumo, smodo, h2, k=k, elu=True)
    return x
